# baseline (speedup 1.0000x reference)
import functools

import jax
import jax.numpy as jnp
from jax import lax
from jax.experimental import pallas as pl
from jax.experimental.pallas import tpu as pltpu
from jax.experimental.pallas import tpu_sc as plsc

OUT_DIM = 32
N_S, N_B, N_C = 5, 4096, 26
N_IDX = N_S * N_B * N_C
NUM_WORKERS = 32
PER_WORKER = N_IDX // NUM_WORKERS
GCH = 1280
NCH = PER_WORKER // GCH
BB = 256
PIECES = GCH // BB
UPW = PER_WORKER // BB

_mesh = plsc.VectorSubcoreMesh(core_axis_name="c", subcore_axis_name="s")


@functools.partial(
    pl.kernel,
    mesh=_mesh,
    compiler_params=pltpu.CompilerParams(use_tc_tiling_on_sc=False),
    out_type=jax.ShapeDtypeStruct((N_S * N_B, N_C * OUT_DIM), jnp.float32),
    scratch_types=[
        pltpu.VMEM((PER_WORKER,), jnp.int32),
        pltpu.VMEM((GCH, OUT_DIM), jnp.float32),
        pltpu.VMEM((GCH, OUT_DIM), jnp.float32),
        pltpu.SemaphoreType.DMA,
        pltpu.SemaphoreType.DMA,
    ],
)
def _gather_sc(idx_hbm, table_hbm, out_hbm, idx_v, rows0, rows1, sem0, sem1):
    wid = lax.axis_index("s") * 2 + lax.axis_index("c")
    base = wid * PER_WORKER
    pltpu.sync_copy(idx_hbm.at[pl.ds(base, PER_WORKER)], idx_v)

    bufs = (rows0, rows1)
    sems = (sem0, sem1)

    def issue(ch):
        return pltpu.async_copy(
            table_hbm.at[idx_v.at[pl.ds(ch * GCH, GCH)]],
            bufs[ch % 2], sems[ch % 2])

    d = issue(0)
    for ch in range(NCH):
        d.wait()
        if ch + 1 < NCH:
            d = issue(ch + 1)
        buf = bufs[ch % 2]
        for k in range(PIECES):
            g = wid * UPW + ch * PIECES + k
            p = g // 16
            b0 = (g % 16) * BB
            s_ = ((p >= 26).astype(jnp.int32) + (p >= 52).astype(jnp.int32)
                  + (p >= 78).astype(jnp.int32) + (p >= 104).astype(jnp.int32))
            c_ = p - s_ * 26
            row0 = s_ * N_B + b0
            pltpu.sync_copy(
                buf.at[pl.ds(k * BB, BB)],
                out_hbm.at[pl.ds(row0, BB), pl.ds(c_ * OUT_DIM, OUT_DIM)])


_TB = 65536
_NCAT = 1000000
_TGRID = (_NCAT + _TB - 1) // _TB
_TROWS = _TGRID * (_TB // 4)


def _trans_pen_body(w_ref, out_ref, pen_ref):
    i = pl.program_id(0)
    blk = w_ref[...]
    qs = _TB // 4
    rowb = jax.lax.broadcasted_iota(jnp.int32, (32, 128), 0)
    colb = jax.lax.broadcasted_iota(jnp.int32, (32, 128), 1)
    acc = None
    for q in range(4):
        eq = jnp.float32(colb == rowb + q * 32)
        part = jax.lax.dot_general(blk[:, q * qs:(q + 1) * qs], eq,
                                   (((0,), (0,)), ((), ())),
                                   preferred_element_type=jnp.float32)
        acc = part if acc is None else acc + part
    out_ref[...] = acc

    def masked(b):
        col = jax.lax.broadcasted_iota(jnp.int32, b.shape, 1) + i * _TB
        return jnp.where(col < _NCAT, b, 0.0)

    v = lax.cond(i == _TGRID - 1, masked, lambda b: b, blk)
    s = 0.5 * jnp.sum(v * v) + jnp.sum(jnp.abs(v))

    @pl.when(i == 0)
    def _():
        pen_ref[0, 0] = 0.0

    pen_ref[0, 0] += s


def _transpose_penalty_tc(wt):
    return pl.pallas_call(
        _trans_pen_body,
        grid=(_TGRID,),
        in_specs=[pl.BlockSpec((32, _TB), lambda i: (0, i))],
        out_specs=[
            pl.BlockSpec((_TB // 4, 128), lambda i: (i, 0)),
            pl.BlockSpec(memory_space=pltpu.SMEM),
        ],
        out_shape=[
            jax.ShapeDtypeStruct((_TROWS, 128), jnp.float32),
            jax.ShapeDtypeStruct((1, 1), jnp.float32),
        ],
    )(wt)


def kernel(X, W):
    n_samples, n_batch, input_dim = X.shape
    idxp = X.transpose(0, 2, 1).reshape(-1)
    idxv = ((idxp & ~jnp.int32(_TB - 1)) + ((idxp & (_TB // 4 - 1)) << 2)
            + ((idxp & (_TB - 1)) >> (_TB.bit_length() - 3)))
    table2, pen = _transpose_penalty_tc(W.T)
    out2 = _gather_sc(idxv, table2.reshape(_TROWS * 4, OUT_DIM))
    net = out2.reshape(n_samples, n_batch, input_dim * OUT_DIM)
    return net, pen[0, 0]


# speedup vs baseline: 1.0419x; 1.0419x over previous
import functools

import jax
import jax.numpy as jnp
from jax import lax
from jax.experimental import pallas as pl
from jax.experimental.pallas import tpu as pltpu
from jax.experimental.pallas import tpu_sc as plsc

OUT_DIM = 32
N_S, N_B, N_C = 5, 4096, 26
N_IDX = N_S * N_B * N_C
NUM_WORKERS = 32
PER_WORKER = N_IDX // NUM_WORKERS
GCH = 1280
NCH = PER_WORKER // GCH
BB = 256
PIECES = GCH // BB
UPW = PER_WORKER // BB

_mesh = plsc.VectorSubcoreMesh(core_axis_name="c", subcore_axis_name="s")


@functools.partial(
    pl.kernel,
    mesh=_mesh,
    compiler_params=pltpu.CompilerParams(use_tc_tiling_on_sc=False),
    out_type=jax.ShapeDtypeStruct((N_S * N_B, N_C * OUT_DIM), jnp.float32),
    scratch_types=[
        pltpu.VMEM((PER_WORKER,), jnp.int32),
        pltpu.VMEM((GCH, OUT_DIM), jnp.float32),
        pltpu.VMEM((GCH, OUT_DIM), jnp.float32),
        pltpu.SemaphoreType.DMA,
        pltpu.SemaphoreType.DMA,
    ],
)
def _gather_sc(idx_hbm, table_hbm, out_hbm, idx_v, rows0, rows1, sem0, sem1):
    wid = lax.axis_index("s") * 2 + lax.axis_index("c")
    base = wid * PER_WORKER
    pltpu.sync_copy(idx_hbm.at[pl.ds(base, PER_WORKER)], idx_v)

    bufs = (rows0, rows1)
    sems = (sem0, sem1)

    def issue(ch):
        return pltpu.async_copy(
            table_hbm.at[idx_v.at[pl.ds(ch * GCH, GCH)]],
            bufs[ch % 2], sems[ch % 2])

    d = issue(0)
    for ch in range(NCH):
        d.wait()
        if ch + 1 < NCH:
            d = issue(ch + 1)
        buf = bufs[ch % 2]
        for k in range(PIECES):
            g = wid * UPW + ch * PIECES + k
            p = g // 16
            b0 = (g % 16) * BB
            s_ = ((p >= 26).astype(jnp.int32) + (p >= 52).astype(jnp.int32)
                  + (p >= 78).astype(jnp.int32) + (p >= 104).astype(jnp.int32))
            c_ = p - s_ * 26
            row0 = s_ * N_B + b0
            pltpu.sync_copy(
                buf.at[pl.ds(k * BB, BB)],
                out_hbm.at[pl.ds(row0, BB), pl.ds(c_ * OUT_DIM, OUT_DIM)])


_TB = 32768
_NCAT = 1000000
_TGRID = (_NCAT + _TB - 1) // _TB
_TROWS = _TGRID * (_TB // 4)


def _trans_pen_body(w_ref, out_ref):
    blk = w_ref[...]
    qs = _TB // 4
    rowb = jax.lax.broadcasted_iota(jnp.int32, (32, 128), 0)
    colb = jax.lax.broadcasted_iota(jnp.int32, (32, 128), 1)
    acc = None
    for q in range(4):
        eq = jnp.float32(colb == rowb + q * 32)
        part = jax.lax.dot_general(blk[:, q * qs:(q + 1) * qs], eq,
                                   (((0,), (0,)), ((), ())),
                                   preferred_element_type=jnp.float32)
        acc = part if acc is None else acc + part
    out_ref[...] = acc


def _transpose_tc(wt):
    return pl.pallas_call(
        _trans_pen_body,
        grid=(_TGRID,),
        in_specs=[pl.BlockSpec((32, _TB), lambda i: (0, i))],
        out_specs=pl.BlockSpec((_TB // 4, 128), lambda i: (i, 0)),
        out_shape=jax.ShapeDtypeStruct((_TROWS, 128), jnp.float32),
    )(wt)


_PEN_BLOCK = 65536


def _penalty_body(w_ref, out_ref):
    i = pl.program_id(0)
    nblk = pl.num_programs(0)
    blk = w_ref[...]

    def masked(b):
        col = jax.lax.broadcasted_iota(jnp.int32, b.shape, 1) + i * _PEN_BLOCK
        return jnp.where(col < _NCAT, b, 0.0)

    v = lax.cond(i == nblk - 1, masked, lambda b: b, blk)
    s = 0.5 * jnp.sum(v * v) + jnp.sum(jnp.abs(v))

    @pl.when(i == 0)
    def _():
        out_ref[0, 0] = 0.0

    out_ref[0, 0] += s


def _penalty_tc(wt):
    d, ncat = wt.shape
    nblk = (ncat + _PEN_BLOCK - 1) // _PEN_BLOCK
    return pl.pallas_call(
        _penalty_body,
        grid=(nblk,),
        in_specs=[pl.BlockSpec((d, _PEN_BLOCK), lambda i: (0, i))],
        out_specs=pl.BlockSpec(memory_space=pltpu.SMEM),
        out_shape=jax.ShapeDtypeStruct((1, 1), jnp.float32),
    )(wt)


def kernel(X, W):
    n_samples, n_batch, input_dim = X.shape
    idxp = X.transpose(0, 2, 1).reshape(-1)
    idxv = ((idxp & ~jnp.int32(_TB - 1)) + ((idxp & (_TB // 4 - 1)) << 2)
            + ((idxp & (_TB - 1)) >> (_TB.bit_length() - 3)))
    wt = W.T
    table2 = _transpose_tc(wt)
    out2 = _gather_sc(idxv, table2.reshape(_TROWS * 4, OUT_DIM))
    net = out2.reshape(n_samples, n_batch, input_dim * OUT_DIM)
    pen = _penalty_tc(wt)[0, 0]
    return net, pen
